# baseline (speedup 1.0000x reference)
import functools

import jax
import jax.numpy as jnp
from jax.experimental import pallas as pl
from jax.experimental.pallas import tpu as pltpu


def _mlp_block(ef_ref, w1_ref, b1_ref, w2_ref, b2_ref, w3_ref, b3_ref,
               w4_ref, b4_ref, out_ref):
    o = ef_ref[...]
    o = jnp.dot(o, w1_ref[...], preferred_element_type=jnp.float32) + b1_ref[...]
    o = jnp.where(o >= 0, o, 0.01 * o)
    o = jnp.dot(o, w2_ref[...], preferred_element_type=jnp.float32) + b2_ref[...]
    o = jnp.where(o >= 0, o, 0.01 * o)
    o = jnp.dot(o, w3_ref[...], preferred_element_type=jnp.float32) + b3_ref[...]
    o = jnp.where(o >= 0, o, 0.01 * o)
    o = jnp.dot(o, w4_ref[...], preferred_element_type=jnp.float32) + b4_ref[...]
    out_ref[...] = o


def _edge_mlp(ef, fc1_W, fc1_b, fc2_W, fc2_b, fc3_W, fc3_b, fc4_W, fc4_b):
    R = ef.shape[0]
    BR = 1024
    grid = (R // BR,)
    full = lambda *s: pl.BlockSpec(s, lambda i: (0,) * len(s))
    return pl.pallas_call(
        _mlp_block,
        grid=grid,
        in_specs=[
            pl.BlockSpec((BR, 512), lambda i: (i, 0)),
            full(512, 256), full(1, 256),
            full(256, 128), full(1, 128),
            full(128, 128), full(1, 128),
            full(128, 1), full(1, 1),
        ],
        out_specs=pl.BlockSpec((BR, 1), lambda i: (i, 0)),
        out_shape=jax.ShapeDtypeStruct((R, 1), jnp.float32),
    )(ef, fc1_W, fc1_b.reshape(1, -1), fc2_W, fc2_b.reshape(1, -1),
      fc3_W, fc3_b.reshape(1, -1), fc4_W, fc4_b.reshape(1, -1))


def kernel(x, edge_index, pair_src, pair_dst, W1, b1, g1, be1, W2, b2, g2, be2,
           W3, b3, g3, be3, fc1_W, fc1_b, fc2_W, fc2_b, fc3_W, fc3_b, fc4_W, fc4_b):
    N = x.shape[0]
    num_nodes = 30
    src = edge_index[0]
    dst = edge_index[1]
    deg = jnp.zeros((N,), x.dtype).at[src].add(1.0)
    dinv = jnp.where(deg > 0, 1.0 / jnp.sqrt(jnp.maximum(deg, 1e-12)), 0.0)
    w = -dinv[src] * dinv[dst]

    def prop(h):
        return jnp.zeros_like(h).at[dst].add(w[:, None] * h[src])

    def cheb(h, W, b):
        K = W.shape[0]
        Tx0 = h
        out = Tx0 @ W[0]
        Tx1 = prop(h)
        out = out + Tx1 @ W[1]
        for k in range(2, K):
            Tx2 = 2.0 * prop(Tx1) - Tx0
            out = out + Tx2 @ W[k]
            Tx0, Tx1 = Tx1, Tx2
        return out + b

    def lrelu(h):
        return jnp.where(h >= 0, h, 0.01 * h)

    def ln(h, g, b):
        mu = h.mean(-1, keepdims=True)
        var = h.var(-1, keepdims=True)
        return (h - mu) / jnp.sqrt(var + 1e-5) * g + b

    h = ln(lrelu(cheb(x, W1, b1)), g1, be1)
    h = ln(lrelu(cheb(h, W2, b2)), g2, be2)
    h = ln(lrelu(cheb(h, W3, b3)), g3, be3)
    B = N // num_nodes
    h = h.reshape(B, num_nodes, -1)
    ef = jnp.concatenate([h[:, pair_src, :], h[:, pair_dst, :]], axis=-1)
    ef = ef.reshape(-1, ef.shape[-1])
    o = _edge_mlp(ef, fc1_W, fc1_b, fc2_W, fc2_b, fc3_W, fc3_b, fc4_W, fc4_b)
    return o.reshape(-1)


# speedup vs baseline: 5.7427x; 5.7427x over previous
import functools

import jax
import jax.numpy as jnp
from jax import lax
from jax.experimental import pallas as pl
from jax.experimental.pallas import tpu as pltpu
from jax.experimental.pallas import tpu_sc as plsc

N_NODES = 30720
E_EDGES = 491520
N_TILES = 16
N_CORES = 2
EC = 128
EPT = E_EDGES // N_TILES
NCH = EPT // EC
IBLK = 120
NBLK = NCH // IBLK
RPT = N_NODES // N_TILES
FC = 32


def _sc_prop_body(C, u_hbm, src_hbm, dst_hbm, r_hbm,
                  acc, src_idx, dst_idx, rows, zbuf, sems):
    cid = lax.axis_index("c")
    sid = lax.axis_index("s")

    def _zb(i, _):
        for j in range(FC // 16):
            zbuf[i, pl.ds(j * 16, 16)] = jnp.zeros((16,), jnp.float32)
        return 0
    lax.fori_loop(0, 128, _zb, 0)

    rowbase = sid * RPT
    for rr in range(C // N_CORES):
        chunk = rr * N_CORES + cid
        uv = u_hbm.at[chunk]
        rv = r_hbm.at[chunk]

        def _zero(m, _):
            pltpu.sync_copy(zbuf, acc.at[pl.ds(rowbase + m * 128, 128)])
            return 0
        lax.fori_loop(0, RPT // 128, _zero, 0)
        plsc.subcore_barrier()

        for blk in range(NBLK):
            pltpu.sync_copy(src_hbm.at[sid, pl.ds(blk * IBLK, IBLK)], src_idx)
            pltpu.sync_copy(dst_hbm.at[sid, pl.ds(blk * IBLK, IBLK)], dst_idx)
            pltpu.async_copy(uv.at[src_idx.at[0]], rows.at[0], sems.at[0])
            pltpu.async_copy(uv.at[src_idx.at[1]], rows.at[1], sems.at[1])

            def _body(g, _):
                for b in range(2):
                    j = 2 * g + b
                    pltpu.make_async_copy(
                        uv.at[src_idx.at[j]], rows.at[b], sems.at[b]).wait()
                    pltpu.sync_copy(rows.at[b], acc.at[dst_idx.at[j]], add=True)
                    nxt = j + 2

                    @pl.when(nxt < IBLK)
                    def _():
                        pltpu.async_copy(
                            uv.at[src_idx.at[nxt]], rows.at[b], sems.at[b])
                return 0
            lax.fori_loop(0, IBLK // 2, _body, 0)
        plsc.subcore_barrier()

        def _out(m, _):
            sl = pl.ds(rowbase + m * 128, 128)
            pltpu.sync_copy(acc.at[sl], rows.at[0])
            pltpu.sync_copy(rows.at[0], rv.at[sl])
            return 0
        lax.fori_loop(0, RPT // 128, _out, 0)
        plsc.subcore_barrier()


@functools.partial(jax.jit, static_argnums=(3,))
def _sc_prop(u3, src3, dst3, C):
    mesh = plsc.VectorSubcoreMesh(core_axis_name="c", subcore_axis_name="s")
    return pl.kernel(
        functools.partial(_sc_prop_body, C),
        out_type=jax.ShapeDtypeStruct((C, N_NODES, FC), jnp.float32),
        mesh=mesh,
        compiler_params=pltpu.CompilerParams(use_tc_tiling_on_sc=False),
        scratch_types=[
            pltpu.VMEM_SHARED((N_NODES, FC), jnp.float32),
            pltpu.VMEM((IBLK, EC), jnp.int32),
            pltpu.VMEM((IBLK, EC), jnp.int32),
            pltpu.VMEM((2, EC, FC), jnp.float32),
            pltpu.VMEM((128, FC), jnp.float32),
            pltpu.SemaphoreType.DMA((2,)),
        ],
    )(u3, src3, dst3)


def _R(u, src3, dst3):
    C = u.shape[1] // FC
    u3 = u.reshape(N_NODES, C, FC).transpose(1, 0, 2)
    r3 = _sc_prop(u3, src3, dst3, C)
    return r3.transpose(1, 0, 2).reshape(N_NODES, C * FC)


def _mlp_block(ef_ref, w1_ref, b1_ref, w2_ref, b2_ref, w3_ref, b3_ref,
               w4_ref, b4_ref, out_ref):
    o = ef_ref[...]
    o = jnp.dot(o, w1_ref[...], preferred_element_type=jnp.float32) + b1_ref[...]
    o = jnp.where(o >= 0, o, 0.01 * o)
    o = jnp.dot(o, w2_ref[...], preferred_element_type=jnp.float32) + b2_ref[...]
    o = jnp.where(o >= 0, o, 0.01 * o)
    o = jnp.dot(o, w3_ref[...], preferred_element_type=jnp.float32) + b3_ref[...]
    o = jnp.where(o >= 0, o, 0.01 * o)
    o = jnp.dot(o, w4_ref[...], preferred_element_type=jnp.float32) + b4_ref[...]
    out_ref[...] = o


def _edge_mlp(ef, fc1_W, fc1_b, fc2_W, fc2_b, fc3_W, fc3_b, fc4_W, fc4_b):
    R = ef.shape[0]
    BR = 1024
    grid = (R // BR,)
    full = lambda *s: pl.BlockSpec(s, lambda i: (0,) * len(s))
    return pl.pallas_call(
        _mlp_block,
        grid=grid,
        in_specs=[
            pl.BlockSpec((BR, 512), lambda i: (i, 0)),
            full(512, 256), full(1, 256),
            full(256, 128), full(1, 128),
            full(128, 128), full(1, 128),
            full(128, 1), full(1, 1),
        ],
        out_specs=pl.BlockSpec((BR, 1), lambda i: (i, 0)),
        out_shape=jax.ShapeDtypeStruct((R, 1), jnp.float32),
    )(ef, fc1_W, fc1_b.reshape(1, -1), fc2_W, fc2_b.reshape(1, -1),
      fc3_W, fc3_b.reshape(1, -1), fc4_W, fc4_b.reshape(1, -1))


def kernel(x, edge_index, pair_src, pair_dst, W1, b1, g1, be1, W2, b2, g2, be2,
           W3, b3, g3, be3, fc1_W, fc1_b, fc2_W, fc2_b, fc3_W, fc3_b, fc4_W, fc4_b):
    N = x.shape[0]
    num_nodes = 30
    src = edge_index[0]
    dst = edge_index[1]
    src3 = src.reshape(N_TILES, NCH, EC)
    dst3 = dst.reshape(N_TILES, NCH, EC)

    deg = jnp.zeros((N,), x.dtype).at[src].add(1.0)
    dinv = jnp.where(deg > 0, 1.0 / jnp.sqrt(jnp.maximum(deg, 1e-12)), 0.0)
    s = dinv[:, None]

    def prop(h):
        return -s * _R(s * h, src3, dst3)

    def cheb(h, W, b):
        K = W.shape[0]
        Tx0 = h
        out = Tx0 @ W[0]
        Tx1 = prop(h)
        out = out + Tx1 @ W[1]
        for k in range(2, K):
            Tx2 = 2.0 * prop(Tx1) - Tx0
            out = out + Tx2 @ W[k]
            Tx0, Tx1 = Tx1, Tx2
        return out + b

    def lrelu(h):
        return jnp.where(h >= 0, h, 0.01 * h)

    def ln(h, g, b):
        mu = h.mean(-1, keepdims=True)
        var = h.var(-1, keepdims=True)
        return (h - mu) / jnp.sqrt(var + 1e-5) * g + b

    h = ln(lrelu(cheb(x, W1, b1)), g1, be1)
    h = ln(lrelu(cheb(h, W2, b2)), g2, be2)
    h = ln(lrelu(cheb(h, W3, b3)), g3, be3)
    B = N // num_nodes
    h = h.reshape(B, num_nodes, -1)
    ef = jnp.concatenate([h[:, pair_src, :], h[:, pair_dst, :]], axis=-1)
    ef = ef.reshape(-1, ef.shape[-1])
    o = _edge_mlp(ef, fc1_W, fc1_b, fc2_W, fc2_b, fc3_W, fc3_b, fc4_W, fc4_b)
    return o.reshape(-1)


# speedup vs baseline: 6.8240x; 1.1883x over previous
import functools

import jax
import jax.numpy as jnp
from jax import lax
from jax.experimental import pallas as pl
from jax.experimental.pallas import tpu as pltpu
from jax.experimental.pallas import tpu_sc as plsc

N_NODES = 30720
E_EDGES = 491520
N_TILES = 16
N_CORES = 2
EC = 128
EPT = E_EDGES // N_TILES
NCH = EPT // EC
IBLK = 40
NBLK = NCH // IBLK
NSLOT = 8
LOOKAHEAD = 4
RPT = N_NODES // N_TILES
FC = 32


def _sc_prop_body(C, u_hbm, src_hbm, dst_hbm, r_hbm,
                  acc, src_idx, dst_idx, rows, zbuf, gsems, ssems):
    cid = lax.axis_index("c")
    sid = lax.axis_index("s")

    def _zb(i, _):
        for j in range(FC // 16):
            zbuf[i, pl.ds(j * 16, 16)] = jnp.zeros((16,), jnp.float32)
        return 0
    lax.fori_loop(0, 128, _zb, 0)

    rowbase = sid * RPT
    for rr in range(C // N_CORES):
        chunk = rr * N_CORES + cid
        uv = u_hbm.at[chunk]
        rv = r_hbm.at[chunk]

        def _zero(m, _):
            pltpu.sync_copy(zbuf, acc.at[pl.ds(rowbase + m * 128, 128)])
            return 0
        lax.fori_loop(0, RPT // 128, _zero, 0)
        plsc.subcore_barrier()

        for blk in range(NBLK):
            pltpu.sync_copy(src_hbm.at[sid, pl.ds(blk * IBLK, IBLK)], src_idx)
            pltpu.sync_copy(dst_hbm.at[sid, pl.ds(blk * IBLK, IBLK)], dst_idx)
            for b in range(LOOKAHEAD):
                pltpu.async_copy(uv.at[src_idx.at[b]], rows.at[b], gsems.at[b])

            def _body(g, _):
                for u in range(NSLOT):
                    j = g * NSLOT + u
                    b = u
                    pltpu.make_async_copy(
                        uv.at[src_idx.at[j]], rows.at[b], gsems.at[b]).wait()
                    pltpu.async_copy(rows.at[b], acc.at[dst_idx.at[j]],
                                     ssems.at[b], add=True)
                    nxt = j + LOOKAHEAD
                    nb = (u + LOOKAHEAD) % NSLOT

                    @pl.when(nxt < IBLK)
                    def _():
                        @pl.when(j >= LOOKAHEAD)
                        def _():
                            pltpu.make_async_copy(
                                rows.at[nb],
                                acc.at[dst_idx.at[j - LOOKAHEAD]],
                                ssems.at[nb]).wait()
                        pltpu.async_copy(uv.at[src_idx.at[nxt]], rows.at[nb],
                                         gsems.at[nb])
                return 0
            lax.fori_loop(0, IBLK // NSLOT, _body, 0)
            for k in range(NSLOT):
                j = IBLK - NSLOT + k
                b = j % NSLOT
                pltpu.make_async_copy(
                    rows.at[b], acc.at[dst_idx.at[j]], ssems.at[b]).wait()
        plsc.subcore_barrier()

        def _out(m, _):
            sl = pl.ds(rowbase + m * 128, 128)
            pltpu.sync_copy(acc.at[sl], rows.at[0])
            pltpu.sync_copy(rows.at[0], rv.at[sl])
            return 0
        lax.fori_loop(0, RPT // 128, _out, 0)
        plsc.subcore_barrier()


@functools.partial(jax.jit, static_argnums=(3,))
def _sc_prop(u3, src3, dst3, C):
    mesh = plsc.VectorSubcoreMesh(core_axis_name="c", subcore_axis_name="s")
    return pl.kernel(
        functools.partial(_sc_prop_body, C),
        out_type=jax.ShapeDtypeStruct((C, N_NODES, FC), jnp.float32),
        mesh=mesh,
        compiler_params=pltpu.CompilerParams(use_tc_tiling_on_sc=False),
        scratch_types=[
            pltpu.VMEM_SHARED((N_NODES, FC), jnp.float32),
            pltpu.VMEM((IBLK, EC), jnp.int32),
            pltpu.VMEM((IBLK, EC), jnp.int32),
            pltpu.VMEM((NSLOT, EC, FC), jnp.float32),
            pltpu.VMEM((128, FC), jnp.float32),
            pltpu.SemaphoreType.DMA((NSLOT,)),
            pltpu.SemaphoreType.DMA((NSLOT,)),
        ],
    )(u3, src3, dst3)


def _R(u, src3, dst3):
    C = u.shape[1] // FC
    u3 = u.reshape(N_NODES, C, FC).transpose(1, 0, 2)
    r3 = _sc_prop(u3, src3, dst3, C)
    return r3.transpose(1, 0, 2).reshape(N_NODES, C * FC)


def _mlp_block(ef_ref, w1_ref, b1_ref, w2_ref, b2_ref, w3_ref, b3_ref,
               w4_ref, b4_ref, out_ref):
    o = ef_ref[...]
    o = jnp.dot(o, w1_ref[...], preferred_element_type=jnp.float32) + b1_ref[...]
    o = jnp.where(o >= 0, o, 0.01 * o)
    o = jnp.dot(o, w2_ref[...], preferred_element_type=jnp.float32) + b2_ref[...]
    o = jnp.where(o >= 0, o, 0.01 * o)
    o = jnp.dot(o, w3_ref[...], preferred_element_type=jnp.float32) + b3_ref[...]
    o = jnp.where(o >= 0, o, 0.01 * o)
    o = jnp.dot(o, w4_ref[...], preferred_element_type=jnp.float32) + b4_ref[...]
    out_ref[...] = o


def _edge_mlp(ef, fc1_W, fc1_b, fc2_W, fc2_b, fc3_W, fc3_b, fc4_W, fc4_b):
    R = ef.shape[0]
    BR = 1024
    grid = (R // BR,)
    full = lambda *s: pl.BlockSpec(s, lambda i: (0,) * len(s))
    return pl.pallas_call(
        _mlp_block,
        grid=grid,
        in_specs=[
            pl.BlockSpec((BR, 512), lambda i: (i, 0)),
            full(512, 256), full(1, 256),
            full(256, 128), full(1, 128),
            full(128, 128), full(1, 128),
            full(128, 1), full(1, 1),
        ],
        out_specs=pl.BlockSpec((BR, 1), lambda i: (i, 0)),
        out_shape=jax.ShapeDtypeStruct((R, 1), jnp.float32),
    )(ef, fc1_W, fc1_b.reshape(1, -1), fc2_W, fc2_b.reshape(1, -1),
      fc3_W, fc3_b.reshape(1, -1), fc4_W, fc4_b.reshape(1, -1))


def kernel(x, edge_index, pair_src, pair_dst, W1, b1, g1, be1, W2, b2, g2, be2,
           W3, b3, g3, be3, fc1_W, fc1_b, fc2_W, fc2_b, fc3_W, fc3_b, fc4_W, fc4_b):
    N = x.shape[0]
    num_nodes = 30
    src = edge_index[0]
    dst = edge_index[1]
    src3 = src.reshape(N_TILES, NCH, EC)
    dst3 = dst.reshape(N_TILES, NCH, EC)

    deg = jnp.zeros((N,), x.dtype).at[src].add(1.0)
    dinv = jnp.where(deg > 0, 1.0 / jnp.sqrt(jnp.maximum(deg, 1e-12)), 0.0)
    s = dinv[:, None]

    def prop(h):
        return -s * _R(s * h, src3, dst3)

    def cheb(h, W, b):
        K = W.shape[0]
        Tx0 = h
        out = Tx0 @ W[0]
        Tx1 = prop(h)
        out = out + Tx1 @ W[1]
        for k in range(2, K):
            Tx2 = 2.0 * prop(Tx1) - Tx0
            out = out + Tx2 @ W[k]
            Tx0, Tx1 = Tx1, Tx2
        return out + b

    def lrelu(h):
        return jnp.where(h >= 0, h, 0.01 * h)

    def ln(h, g, b):
        mu = h.mean(-1, keepdims=True)
        var = h.var(-1, keepdims=True)
        return (h - mu) / jnp.sqrt(var + 1e-5) * g + b

    h = ln(lrelu(cheb(x, W1, b1)), g1, be1)
    h = ln(lrelu(cheb(h, W2, b2)), g2, be2)
    h = ln(lrelu(cheb(h, W3, b3)), g3, be3)
    B = N // num_nodes
    h = h.reshape(B, num_nodes, -1)
    ef = jnp.concatenate([h[:, pair_src, :], h[:, pair_dst, :]], axis=-1)
    ef = ef.reshape(-1, ef.shape[-1])
    o = _edge_mlp(ef, fc1_W, fc1_b, fc2_W, fc2_b, fc3_W, fc3_b, fc4_W, fc4_b)
    return o.reshape(-1)


# speedup vs baseline: 7.3341x; 1.0747x over previous
import functools

import jax
import jax.numpy as jnp
from jax import lax
from jax.experimental import pallas as pl
from jax.experimental.pallas import tpu as pltpu
from jax.experimental.pallas import tpu_sc as plsc

N_NODES = 30720
E_EDGES = 491520
N_TILES = 16
N_CORES = 2
EC = 128
EPT = E_EDGES // N_TILES
NCH = EPT // EC
IBLK = 40
NBLK = NCH // IBLK
NSLOT = 8
LOOKAHEAD = 4
RPT = N_NODES // N_TILES
FC = 32


def _sc_prop_body(C, u_hbm, src_hbm, dst_hbm, r_hbm,
                  acc, src_idx, dst_idx, rows, zbuf, gsems, ssems):
    cid = lax.axis_index("c")
    sid = lax.axis_index("s")

    def _zb(i, _):
        for j in range(FC // 16):
            zbuf[i, pl.ds(j * 16, 16)] = jnp.zeros((16,), jnp.float32)
        return 0
    lax.fori_loop(0, 128, _zb, 0)

    rowbase = sid * RPT
    for rr in range(C // N_CORES):
        chunk = rr * N_CORES + cid
        uv = u_hbm.at[chunk]
        rv = r_hbm.at[chunk]

        def _zero(m, _):
            pltpu.sync_copy(zbuf, acc.at[pl.ds(rowbase + m * 128, 128)])
            return 0
        lax.fori_loop(0, RPT // 128, _zero, 0)
        plsc.subcore_barrier()

        for blk in range(NBLK):
            pltpu.sync_copy(src_hbm.at[sid, pl.ds(blk * IBLK, IBLK)], src_idx)
            pltpu.sync_copy(dst_hbm.at[sid, pl.ds(blk * IBLK, IBLK)], dst_idx)
            for b in range(LOOKAHEAD):
                pltpu.async_copy(uv.at[src_idx.at[b]], rows.at[b], gsems.at[b])

            def _body(g, _):
                for u in range(NSLOT):
                    j = g * NSLOT + u
                    b = u
                    pltpu.make_async_copy(
                        uv.at[src_idx.at[j]], rows.at[b], gsems.at[b]).wait()
                    pltpu.async_copy(rows.at[b], acc.at[dst_idx.at[j]],
                                     ssems.at[b], add=True)
                    nxt = j + LOOKAHEAD
                    nb = (u + LOOKAHEAD) % NSLOT

                    @pl.when(nxt < IBLK)
                    def _():
                        @pl.when(j >= LOOKAHEAD)
                        def _():
                            pltpu.make_async_copy(
                                rows.at[nb],
                                acc.at[dst_idx.at[j - LOOKAHEAD]],
                                ssems.at[nb]).wait()
                        pltpu.async_copy(uv.at[src_idx.at[nxt]], rows.at[nb],
                                         gsems.at[nb])
                return 0
            lax.fori_loop(0, IBLK // NSLOT, _body, 0)
            for k in range(NSLOT):
                j = IBLK - NSLOT + k
                b = j % NSLOT
                pltpu.make_async_copy(
                    rows.at[b], acc.at[dst_idx.at[j]], ssems.at[b]).wait()
        plsc.subcore_barrier()

        def _out(m, _):
            sl = pl.ds(rowbase + m * 128, 128)
            pltpu.sync_copy(acc.at[sl], rows.at[0])
            pltpu.sync_copy(rows.at[0], rv.at[sl])
            return 0
        lax.fori_loop(0, RPT // 128, _out, 0)
        plsc.subcore_barrier()


@functools.partial(jax.jit, static_argnums=(3,))
def _sc_prop(u3, src3, dst3, C):
    mesh = plsc.VectorSubcoreMesh(core_axis_name="c", subcore_axis_name="s")
    return pl.kernel(
        functools.partial(_sc_prop_body, C),
        out_type=jax.ShapeDtypeStruct((C, N_NODES, FC), jnp.float32),
        mesh=mesh,
        compiler_params=pltpu.CompilerParams(use_tc_tiling_on_sc=False),
        scratch_types=[
            pltpu.VMEM_SHARED((N_NODES, FC), jnp.float32),
            pltpu.VMEM((IBLK, EC), jnp.int32),
            pltpu.VMEM((IBLK, EC), jnp.int32),
            pltpu.VMEM((NSLOT, EC, FC), jnp.float32),
            pltpu.VMEM((128, FC), jnp.float32),
            pltpu.SemaphoreType.DMA((NSLOT,)),
            pltpu.SemaphoreType.DMA((NSLOT,)),
        ],
    )(u3, src3, dst3)


def _R(u, src3, dst3):
    C = u.shape[1] // FC
    u3 = u.reshape(N_NODES, C, FC).transpose(1, 0, 2)
    r3 = _sc_prop(u3, src3, dst3, C)
    return r3.transpose(1, 0, 2).reshape(N_NODES, C * FC)


DEGW = 16
NCH_CORE = NCH // N_CORES


def _sc_deg_body(src_hbm, deg_hbm, acc, idx, ones, ssems):
    cid = lax.axis_index("c")
    sid = lax.axis_index("s")

    def _ob(i, _):
        ones[i, pl.ds(0, 16)] = jnp.full((16,), 1.0, jnp.float32)
        return 0
    lax.fori_loop(0, EC, _ob, 0)

    rowbase = sid * RPT
    plsc.subcore_barrier()

    for blk in range(NCH_CORE // IBLK):
        base = cid * NCH_CORE + blk * IBLK
        pltpu.sync_copy(src_hbm.at[sid, pl.ds(base, IBLK)], idx)

        def _body(g, _):
            for u in range(NSLOT):
                j = g * NSLOT + u
                pltpu.async_copy(ones, acc.at[idx.at[j]], ssems.at[u],
                                 add=True)

                @pl.when(j >= NSLOT)
                def _():
                    pltpu.make_async_copy(
                        ones, acc.at[idx.at[j - NSLOT]], ssems.at[u]).wait()
            return 0
        lax.fori_loop(0, IBLK // NSLOT, _body, 0)
        for k in range(NSLOT):
            j = IBLK - NSLOT + k
            pltpu.make_async_copy(
                ones, acc.at[idx.at[j]], ssems.at[j % NSLOT]).wait()
    plsc.subcore_barrier()

    def _out(m, _):
        sl = pl.ds(rowbase + m * 128, 128)
        pltpu.sync_copy(acc.at[sl], ones)
        pltpu.sync_copy(ones, deg_hbm.at[cid].at[sl])
        return 0
    lax.fori_loop(0, RPT // 128, _out, 0)


@jax.jit
def _sc_deg(src3, zeros16):
    mesh = plsc.VectorSubcoreMesh(core_axis_name="c", subcore_axis_name="s")

    def body(src_hbm, z_hbm, deg_hbm, acc, idx, ones, ssems):
        sid = lax.axis_index("s")
        rowbase = sid * RPT
        pltpu.sync_copy(z_hbm.at[pl.ds(rowbase, RPT)],
                        acc.at[pl.ds(rowbase, RPT)])
        _sc_deg_body(src_hbm, deg_hbm, acc, idx, ones, ssems)

    return pl.kernel(
        body,
        out_type=jax.ShapeDtypeStruct((N_CORES, N_NODES, DEGW), jnp.float32),
        mesh=mesh,
        compiler_params=pltpu.CompilerParams(use_tc_tiling_on_sc=False),
        scratch_types=[
            pltpu.VMEM_SHARED((N_NODES, DEGW), jnp.float32),
            pltpu.VMEM((IBLK, EC), jnp.int32),
            pltpu.VMEM((EC, DEGW), jnp.float32),
            pltpu.SemaphoreType.DMA((NSLOT,)),
        ],
    )(src3, zeros16)


BN = 512
NBLK_TC = N_NODES // BN


def _row_specs(Fin):
    return [pl.BlockSpec((BN, Fin), lambda i: (i, 0))]


def _start_body(h_ref, w_ref, s_ref, acc_ref, u_ref):
    h = h_ref[...]
    acc_ref[...] = jnp.dot(h, w_ref[...], preferred_element_type=jnp.float32)
    u_ref[...] = h * s_ref[...]


def _tc_start(h, W0, s):
    Fin = h.shape[1]
    return pl.pallas_call(
        _start_body,
        grid=(NBLK_TC,),
        in_specs=[
            pl.BlockSpec((BN, Fin), lambda i: (i, 0)),
            pl.BlockSpec((Fin, 256), lambda i: (0, 0)),
            pl.BlockSpec((BN, 1), lambda i: (i, 0)),
        ],
        out_specs=[
            pl.BlockSpec((BN, 256), lambda i: (i, 0)),
            pl.BlockSpec((BN, Fin), lambda i: (i, 0)),
        ],
        out_shape=[
            jax.ShapeDtypeStruct((N_NODES, 256), jnp.float32),
            jax.ShapeDtypeStruct((N_NODES, Fin), jnp.float32),
        ],
    )(h, W0, s)


def _step1_body(r_ref, s_ref, w_ref, acc_ref, acc_out, t_out, u_out):
    t = -(s_ref[...] * r_ref[...])
    acc_out[...] = acc_ref[...] + jnp.dot(
        t, w_ref[...], preferred_element_type=jnp.float32)
    t_out[...] = t
    u_out[...] = t * s_ref[...]


def _stepk_body(r_ref, tp_ref, s_ref, w_ref, acc_ref, acc_out, t_out, u_out):
    t = -2.0 * (s_ref[...] * r_ref[...]) - tp_ref[...]
    acc_out[...] = acc_ref[...] + jnp.dot(
        t, w_ref[...], preferred_element_type=jnp.float32)
    t_out[...] = t
    u_out[...] = t * s_ref[...]


def _tc_step(r, tprev, s, Wk, acc):
    Fin = r.shape[1]
    row = lambda F: pl.BlockSpec((BN, F), lambda i: (i, 0))
    ins = [r] if tprev is None else [r, tprev]
    in_specs = [row(Fin)] * len(ins) + [
        pl.BlockSpec((BN, 1), lambda i: (i, 0)),
        pl.BlockSpec((Fin, 256), lambda i: (0, 0)),
        row(256),
    ]
    body = _step1_body if tprev is None else _stepk_body
    return pl.pallas_call(
        body,
        grid=(NBLK_TC,),
        in_specs=in_specs,
        out_specs=[row(256), row(Fin), row(Fin)],
        out_shape=[
            jax.ShapeDtypeStruct((N_NODES, 256), jnp.float32),
            jax.ShapeDtypeStruct((N_NODES, Fin), jnp.float32),
            jax.ShapeDtypeStruct((N_NODES, Fin), jnp.float32),
        ],
    )(*ins, s, Wk, acc)


def _final_body(r_ref, tp_ref, s_ref, w_ref, acc_ref, b_ref, g_ref, be_ref,
                h_out):
    t = -2.0 * (s_ref[...] * r_ref[...]) - tp_ref[...]
    o = acc_ref[...] + jnp.dot(
        t, w_ref[...], preferred_element_type=jnp.float32) + b_ref[...]
    o = jnp.where(o >= 0, o, 0.01 * o)
    mu = jnp.mean(o, axis=-1, keepdims=True)
    var = jnp.mean((o - mu) ** 2, axis=-1, keepdims=True)
    h_out[...] = (o - mu) / jnp.sqrt(var + 1e-5) * g_ref[...] + be_ref[...]


def _tc_final(r, tprev, s, Wk, acc, b, g, be):
    Fin = r.shape[1]
    row = lambda F: pl.BlockSpec((BN, F), lambda i: (i, 0))
    vec = pl.BlockSpec((1, 256), lambda i: (0, 0))
    return pl.pallas_call(
        _final_body,
        grid=(NBLK_TC,),
        in_specs=[row(Fin), row(Fin),
                  pl.BlockSpec((BN, 1), lambda i: (i, 0)),
                  pl.BlockSpec((Fin, 256), lambda i: (0, 0)),
                  row(256), vec, vec, vec],
        out_specs=row(256),
        out_shape=jax.ShapeDtypeStruct((N_NODES, 256), jnp.float32),
    )(r, tprev, s, Wk, acc, b.reshape(1, -1), g.reshape(1, -1),
      be.reshape(1, -1))


def _layer(h, W, b, g, be, s, src3, dst3):
    K = W.shape[0]
    acc, u = _tc_start(h, W[0], s)
    r = _R(u, src3, dst3)
    acc, T1, u = _tc_step(r, None, s, W[1], acc)
    tm2, tm1 = h, T1
    for k in range(2, K):
        r = _R(u, src3, dst3)
        if k < K - 1:
            acc, tk, u = _tc_step(r, tm2, s, W[k], acc)
            tm2, tm1 = tm1, tk
        else:
            return _tc_final(r, tm2, s, W[k], acc, b, g, be)


GB = 32
NUM_PAIRS = 90
NUM_NODES_G = 30


def _mlp_body(h_ref, ps_ref, pd_ref, w1a_ref, w1b_ref, b1_ref, w2_ref,
              b2_ref, w3_ref, b3_ref, w4_ref, b4_ref, out_ref,
              p1_s, p2_s, o1_s):
    hb = h_ref[...].reshape(GB * NUM_NODES_G, 256)
    p1_s[...] = jnp.dot(hb, w1a_ref[...],
                        preferred_element_type=jnp.float32).reshape(
        GB, NUM_NODES_G, 256)
    p2_s[...] = jnp.dot(hb, w1b_ref[...],
                        preferred_element_type=jnp.float32).reshape(
        GB, NUM_NODES_G, 256)
    for j in range(NUM_PAIRS):
        rs = ps_ref[j]
        rd = pd_ref[j]
        e1 = p1_s[:, pl.ds(rs, 1), :].reshape(GB, 256)
        e2 = p2_s[:, pl.ds(rd, 1), :].reshape(GB, 256)
        o1_s[j] = e1 + e2
    o = o1_s[...] + b1_ref[...]
    o = jnp.where(o >= 0, o, 0.01 * o)
    o = o.reshape(NUM_PAIRS * GB, 256)
    o = jnp.dot(o, w2_ref[...], preferred_element_type=jnp.float32) + b2_ref[...]
    o = jnp.where(o >= 0, o, 0.01 * o)
    o = jnp.dot(o, w3_ref[...], preferred_element_type=jnp.float32) + b3_ref[...]
    o = jnp.where(o >= 0, o, 0.01 * o)
    o = jnp.dot(o, w4_ref[...], preferred_element_type=jnp.float32) + b4_ref[...]
    out_ref[...] = o.reshape(NUM_PAIRS, GB).T


def _edge_mlp(h3, pair_src, pair_dst, fc1_W, fc1_b, fc2_W, fc2_b,
              fc3_W, fc3_b, fc4_W, fc4_b):
    B = h3.shape[0]
    grid = (B // GB,)
    full = lambda *sh: pl.BlockSpec(sh, lambda i: (0,) * len(sh))
    out2d = pl.pallas_call(
        _mlp_body,
        grid=grid,
        in_specs=[
            pl.BlockSpec((GB, NUM_NODES_G, 256), lambda i: (i, 0, 0)),
            pl.BlockSpec(memory_space=pltpu.SMEM),
            pl.BlockSpec(memory_space=pltpu.SMEM),
            full(256, 256), full(256, 256), full(1, 256),
            full(256, 128), full(1, 128),
            full(128, 128), full(1, 128),
            full(128, 1), full(1, 1),
        ],
        out_specs=pl.BlockSpec((GB, NUM_PAIRS), lambda i: (i, 0)),
        out_shape=jax.ShapeDtypeStruct((B, NUM_PAIRS), jnp.float32),
        scratch_shapes=[
            pltpu.VMEM((GB, NUM_NODES_G, 256), jnp.float32),
            pltpu.VMEM((GB, NUM_NODES_G, 256), jnp.float32),
            pltpu.VMEM((NUM_PAIRS, GB, 256), jnp.float32),
        ],
    )(h3, pair_src, pair_dst, fc1_W[:256], fc1_W[256:],
      fc1_b.reshape(1, -1), fc2_W, fc2_b.reshape(1, -1),
      fc3_W, fc3_b.reshape(1, -1), fc4_W, fc4_b.reshape(1, -1))
    return out2d.reshape(-1)


def kernel(x, edge_index, pair_src, pair_dst, W1, b1, g1, be1, W2, b2, g2, be2,
           W3, b3, g3, be3, fc1_W, fc1_b, fc2_W, fc2_b, fc3_W, fc3_b, fc4_W, fc4_b):
    src = edge_index[0]
    dst = edge_index[1]
    src3 = src.reshape(N_TILES, NCH, EC)
    dst3 = dst.reshape(N_TILES, NCH, EC)

    zeros16 = jnp.zeros((N_NODES, DEGW), jnp.float32)
    deg_parts = _sc_deg(src3, zeros16)
    deg = deg_parts[0, :, 0] + deg_parts[1, :, 0]
    dinv = jnp.where(deg > 0, 1.0 / jnp.sqrt(jnp.maximum(deg, 1e-12)), 0.0)
    s = dinv[:, None]

    h = _layer(x, W1, b1, g1, be1, s, src3, dst3)
    h = _layer(h, W2, b2, g2, be2, s, src3, dst3)
    h = _layer(h, W3, b3, g3, be3, s, src3, dst3)

    h3 = h.reshape(-1, NUM_NODES_G, 256)
    return _edge_mlp(h3, pair_src, pair_dst, fc1_W, fc1_b, fc2_W, fc2_b,
                     fc3_W, fc3_b, fc4_W, fc4_b)


# speedup vs baseline: 7.4745x; 1.0192x over previous
import functools

import jax
import jax.numpy as jnp
from jax import lax
from jax.experimental import pallas as pl
from jax.experimental.pallas import tpu as pltpu
from jax.experimental.pallas import tpu_sc as plsc

N_NODES = 30720
E_EDGES = 491520
N_TILES = 16
N_CORES = 2
EC = 128
EPT = E_EDGES // N_TILES
NCH = EPT // EC
IBLK = 40
NBLK = NCH // IBLK
NSLOT = 10
LOOKAHEAD = 5
RPT = N_NODES // N_TILES
FC = 32


def _sc_prop_body(C, u_hbm, src_hbm, dst_hbm, r_hbm,
                  acc, src_idx, dst_idx, rows, zbuf, gsems, ssems):
    cid = lax.axis_index("c")
    sid = lax.axis_index("s")

    def _zb(i, _):
        for j in range(FC // 16):
            zbuf[i, pl.ds(j * 16, 16)] = jnp.zeros((16,), jnp.float32)
        return 0
    lax.fori_loop(0, 128, _zb, 0)

    rowbase = sid * RPT
    for rr in range(C // N_CORES):
        chunk = rr * N_CORES + cid
        uv = u_hbm.at[chunk]
        rv = r_hbm.at[chunk]

        def _zero(m, _):
            pltpu.sync_copy(zbuf, acc.at[pl.ds(rowbase + m * 128, 128)])
            return 0
        lax.fori_loop(0, RPT // 128, _zero, 0)
        plsc.subcore_barrier()

        for blk in range(NBLK):
            pltpu.sync_copy(src_hbm.at[sid, pl.ds(blk * IBLK, IBLK)], src_idx)
            pltpu.sync_copy(dst_hbm.at[sid, pl.ds(blk * IBLK, IBLK)], dst_idx)
            for b in range(LOOKAHEAD):
                pltpu.async_copy(uv.at[src_idx.at[b]], rows.at[b], gsems.at[b])

            def _body(g, _):
                for u in range(NSLOT):
                    j = g * NSLOT + u
                    b = u
                    pltpu.make_async_copy(
                        uv.at[src_idx.at[j]], rows.at[b], gsems.at[b]).wait()
                    pltpu.async_copy(rows.at[b], acc.at[dst_idx.at[j]],
                                     ssems.at[b], add=True)
                    nxt = j + LOOKAHEAD
                    nb = (u + LOOKAHEAD) % NSLOT

                    @pl.when(nxt < IBLK)
                    def _():
                        @pl.when(j >= LOOKAHEAD)
                        def _():
                            pltpu.make_async_copy(
                                rows.at[nb],
                                acc.at[dst_idx.at[j - LOOKAHEAD]],
                                ssems.at[nb]).wait()
                        pltpu.async_copy(uv.at[src_idx.at[nxt]], rows.at[nb],
                                         gsems.at[nb])
                return 0
            lax.fori_loop(0, IBLK // NSLOT, _body, 0)
            for k in range(NSLOT):
                j = IBLK - NSLOT + k
                b = j % NSLOT
                pltpu.make_async_copy(
                    rows.at[b], acc.at[dst_idx.at[j]], ssems.at[b]).wait()
        plsc.subcore_barrier()

        def _out(m, _):
            sl = pl.ds(rowbase + m * 128, 128)
            pltpu.sync_copy(acc.at[sl], rows.at[0])
            pltpu.sync_copy(rows.at[0], rv.at[sl])
            return 0
        lax.fori_loop(0, RPT // 128, _out, 0)
        plsc.subcore_barrier()


@functools.partial(jax.jit, static_argnums=(3,))
def _sc_prop(u3, src3, dst3, C):
    mesh = plsc.VectorSubcoreMesh(core_axis_name="c", subcore_axis_name="s")
    return pl.kernel(
        functools.partial(_sc_prop_body, C),
        out_type=jax.ShapeDtypeStruct((C, N_NODES, FC), jnp.float32),
        mesh=mesh,
        compiler_params=pltpu.CompilerParams(use_tc_tiling_on_sc=False),
        scratch_types=[
            pltpu.VMEM_SHARED((N_NODES, FC), jnp.float32),
            pltpu.VMEM((IBLK, EC), jnp.int32),
            pltpu.VMEM((IBLK, EC), jnp.int32),
            pltpu.VMEM((NSLOT, EC, FC), jnp.float32),
            pltpu.VMEM((128, FC), jnp.float32),
            pltpu.SemaphoreType.DMA((NSLOT,)),
            pltpu.SemaphoreType.DMA((NSLOT,)),
        ],
    )(u3, src3, dst3)


def _R(u, src3, dst3):
    C = u.shape[1] // FC
    u3 = u.reshape(N_NODES, C, FC).transpose(1, 0, 2)
    r3 = _sc_prop(u3, src3, dst3, C)
    return r3.transpose(1, 0, 2).reshape(N_NODES, C * FC)


DEGW = 16
NCH_CORE = NCH // N_CORES


def _sc_deg_body(src_hbm, deg_hbm, acc, idx, ones, ssems):
    cid = lax.axis_index("c")
    sid = lax.axis_index("s")

    def _ob(i, _):
        ones[i, pl.ds(0, 16)] = jnp.full((16,), 1.0, jnp.float32)
        return 0
    lax.fori_loop(0, EC, _ob, 0)

    rowbase = sid * RPT
    plsc.subcore_barrier()

    for blk in range(NCH_CORE // IBLK):
        base = cid * NCH_CORE + blk * IBLK
        pltpu.sync_copy(src_hbm.at[sid, pl.ds(base, IBLK)], idx)

        def _body(g, _):
            for u in range(NSLOT):
                j = g * NSLOT + u
                pltpu.async_copy(ones, acc.at[idx.at[j]], ssems.at[u],
                                 add=True)

                @pl.when(j >= NSLOT)
                def _():
                    pltpu.make_async_copy(
                        ones, acc.at[idx.at[j - NSLOT]], ssems.at[u]).wait()
            return 0
        lax.fori_loop(0, IBLK // NSLOT, _body, 0)
        for k in range(NSLOT):
            j = IBLK - NSLOT + k
            pltpu.make_async_copy(
                ones, acc.at[idx.at[j]], ssems.at[j % NSLOT]).wait()
    plsc.subcore_barrier()

    def _out(m, _):
        sl = pl.ds(rowbase + m * 128, 128)
        pltpu.sync_copy(acc.at[sl], ones)
        pltpu.sync_copy(ones, deg_hbm.at[cid].at[sl])
        return 0
    lax.fori_loop(0, RPT // 128, _out, 0)


@jax.jit
def _sc_deg(src3, zeros16):
    mesh = plsc.VectorSubcoreMesh(core_axis_name="c", subcore_axis_name="s")

    def body(src_hbm, z_hbm, deg_hbm, acc, idx, ones, ssems):
        sid = lax.axis_index("s")
        rowbase = sid * RPT
        pltpu.sync_copy(z_hbm.at[pl.ds(rowbase, RPT)],
                        acc.at[pl.ds(rowbase, RPT)])
        _sc_deg_body(src_hbm, deg_hbm, acc, idx, ones, ssems)

    return pl.kernel(
        body,
        out_type=jax.ShapeDtypeStruct((N_CORES, N_NODES, DEGW), jnp.float32),
        mesh=mesh,
        compiler_params=pltpu.CompilerParams(use_tc_tiling_on_sc=False),
        scratch_types=[
            pltpu.VMEM_SHARED((N_NODES, DEGW), jnp.float32),
            pltpu.VMEM((IBLK, EC), jnp.int32),
            pltpu.VMEM((EC, DEGW), jnp.float32),
            pltpu.SemaphoreType.DMA((NSLOT,)),
        ],
    )(src3, zeros16)


BN = 512
NBLK_TC = N_NODES // BN


def _row_specs(Fin):
    return [pl.BlockSpec((BN, Fin), lambda i: (i, 0))]


def _start_body(h_ref, w_ref, s_ref, acc_ref, u_ref):
    h = h_ref[...]
    acc_ref[...] = jnp.dot(h, w_ref[...], preferred_element_type=jnp.float32)
    u_ref[...] = h * s_ref[...]


def _tc_start(h, W0, s):
    Fin = h.shape[1]
    return pl.pallas_call(
        _start_body,
        grid=(NBLK_TC,),
        in_specs=[
            pl.BlockSpec((BN, Fin), lambda i: (i, 0)),
            pl.BlockSpec((Fin, 256), lambda i: (0, 0)),
            pl.BlockSpec((BN, 1), lambda i: (i, 0)),
        ],
        out_specs=[
            pl.BlockSpec((BN, 256), lambda i: (i, 0)),
            pl.BlockSpec((BN, Fin), lambda i: (i, 0)),
        ],
        out_shape=[
            jax.ShapeDtypeStruct((N_NODES, 256), jnp.float32),
            jax.ShapeDtypeStruct((N_NODES, Fin), jnp.float32),
        ],
    )(h, W0, s)


def _step1_body(r_ref, s_ref, w_ref, acc_ref, acc_out, t_out, u_out):
    t = -(s_ref[...] * r_ref[...])
    acc_out[...] = acc_ref[...] + jnp.dot(
        t, w_ref[...], preferred_element_type=jnp.float32)
    t_out[...] = t
    u_out[...] = t * s_ref[...]


def _stepk_body(r_ref, tp_ref, s_ref, w_ref, acc_ref, acc_out, t_out, u_out):
    t = -2.0 * (s_ref[...] * r_ref[...]) - tp_ref[...]
    acc_out[...] = acc_ref[...] + jnp.dot(
        t, w_ref[...], preferred_element_type=jnp.float32)
    t_out[...] = t
    u_out[...] = t * s_ref[...]


def _tc_step(r, tprev, s, Wk, acc):
    Fin = r.shape[1]
    row = lambda F: pl.BlockSpec((BN, F), lambda i: (i, 0))
    ins = [r] if tprev is None else [r, tprev]
    in_specs = [row(Fin)] * len(ins) + [
        pl.BlockSpec((BN, 1), lambda i: (i, 0)),
        pl.BlockSpec((Fin, 256), lambda i: (0, 0)),
        row(256),
    ]
    body = _step1_body if tprev is None else _stepk_body
    return pl.pallas_call(
        body,
        grid=(NBLK_TC,),
        in_specs=in_specs,
        out_specs=[row(256), row(Fin), row(Fin)],
        out_shape=[
            jax.ShapeDtypeStruct((N_NODES, 256), jnp.float32),
            jax.ShapeDtypeStruct((N_NODES, Fin), jnp.float32),
            jax.ShapeDtypeStruct((N_NODES, Fin), jnp.float32),
        ],
    )(*ins, s, Wk, acc)


def _final_body(r_ref, tp_ref, s_ref, w_ref, acc_ref, b_ref, g_ref, be_ref,
                h_out):
    t = -2.0 * (s_ref[...] * r_ref[...]) - tp_ref[...]
    o = acc_ref[...] + jnp.dot(
        t, w_ref[...], preferred_element_type=jnp.float32) + b_ref[...]
    o = jnp.where(o >= 0, o, 0.01 * o)
    mu = jnp.mean(o, axis=-1, keepdims=True)
    var = jnp.mean((o - mu) ** 2, axis=-1, keepdims=True)
    h_out[...] = (o - mu) / jnp.sqrt(var + 1e-5) * g_ref[...] + be_ref[...]


def _tc_final(r, tprev, s, Wk, acc, b, g, be):
    Fin = r.shape[1]
    row = lambda F: pl.BlockSpec((BN, F), lambda i: (i, 0))
    vec = pl.BlockSpec((1, 256), lambda i: (0, 0))
    return pl.pallas_call(
        _final_body,
        grid=(NBLK_TC,),
        in_specs=[row(Fin), row(Fin),
                  pl.BlockSpec((BN, 1), lambda i: (i, 0)),
                  pl.BlockSpec((Fin, 256), lambda i: (0, 0)),
                  row(256), vec, vec, vec],
        out_specs=row(256),
        out_shape=jax.ShapeDtypeStruct((N_NODES, 256), jnp.float32),
    )(r, tprev, s, Wk, acc, b.reshape(1, -1), g.reshape(1, -1),
      be.reshape(1, -1))


def _layer(h, W, b, g, be, s, src3, dst3):
    K = W.shape[0]
    acc, u = _tc_start(h, W[0], s)
    r = _R(u, src3, dst3)
    acc, T1, u = _tc_step(r, None, s, W[1], acc)
    tm2, tm1 = h, T1
    for k in range(2, K):
        r = _R(u, src3, dst3)
        if k < K - 1:
            acc, tk, u = _tc_step(r, tm2, s, W[k], acc)
            tm2, tm1 = tm1, tk
        else:
            return _tc_final(r, tm2, s, W[k], acc, b, g, be)


GB = 32
NUM_PAIRS = 90
NUM_NODES_G = 30


def _mlp_body(h_ref, ps_ref, pd_ref, w1a_ref, w1b_ref, b1_ref, w2_ref,
              b2_ref, w3_ref, b3_ref, w4_ref, b4_ref, out_ref,
              p1_s, p2_s, o1_s):
    hb = h_ref[...].reshape(GB * NUM_NODES_G, 256)
    p1_s[...] = jnp.dot(hb, w1a_ref[...],
                        preferred_element_type=jnp.float32).reshape(
        GB, NUM_NODES_G, 256)
    p2_s[...] = jnp.dot(hb, w1b_ref[...],
                        preferred_element_type=jnp.float32).reshape(
        GB, NUM_NODES_G, 256)
    for j in range(NUM_PAIRS):
        rs = ps_ref[j]
        rd = pd_ref[j]
        e1 = p1_s[:, pl.ds(rs, 1), :].reshape(GB, 256)
        e2 = p2_s[:, pl.ds(rd, 1), :].reshape(GB, 256)
        o1_s[j] = e1 + e2
    o = o1_s[...] + b1_ref[...]
    o = jnp.where(o >= 0, o, 0.01 * o)
    o = o.reshape(NUM_PAIRS * GB, 256)
    o = jnp.dot(o, w2_ref[...], preferred_element_type=jnp.float32) + b2_ref[...]
    o = jnp.where(o >= 0, o, 0.01 * o)
    o = jnp.dot(o, w3_ref[...], preferred_element_type=jnp.float32) + b3_ref[...]
    o = jnp.where(o >= 0, o, 0.01 * o)
    o = jnp.dot(o, w4_ref[...], preferred_element_type=jnp.float32) + b4_ref[...]
    out_ref[...] = o.reshape(NUM_PAIRS, GB).T


def _edge_mlp(h3, pair_src, pair_dst, fc1_W, fc1_b, fc2_W, fc2_b,
              fc3_W, fc3_b, fc4_W, fc4_b):
    B = h3.shape[0]
    grid = (B // GB,)
    full = lambda *sh: pl.BlockSpec(sh, lambda i: (0,) * len(sh))
    out2d = pl.pallas_call(
        _mlp_body,
        grid=grid,
        in_specs=[
            pl.BlockSpec((GB, NUM_NODES_G, 256), lambda i: (i, 0, 0)),
            pl.BlockSpec(memory_space=pltpu.SMEM),
            pl.BlockSpec(memory_space=pltpu.SMEM),
            full(256, 256), full(256, 256), full(1, 256),
            full(256, 128), full(1, 128),
            full(128, 128), full(1, 128),
            full(128, 1), full(1, 1),
        ],
        out_specs=pl.BlockSpec((GB, NUM_PAIRS), lambda i: (i, 0)),
        out_shape=jax.ShapeDtypeStruct((B, NUM_PAIRS), jnp.float32),
        scratch_shapes=[
            pltpu.VMEM((GB, NUM_NODES_G, 256), jnp.float32),
            pltpu.VMEM((GB, NUM_NODES_G, 256), jnp.float32),
            pltpu.VMEM((NUM_PAIRS, GB, 256), jnp.float32),
        ],
    )(h3, pair_src, pair_dst, fc1_W[:256], fc1_W[256:],
      fc1_b.reshape(1, -1), fc2_W, fc2_b.reshape(1, -1),
      fc3_W, fc3_b.reshape(1, -1), fc4_W, fc4_b.reshape(1, -1))
    return out2d.reshape(-1)


def kernel(x, edge_index, pair_src, pair_dst, W1, b1, g1, be1, W2, b2, g2, be2,
           W3, b3, g3, be3, fc1_W, fc1_b, fc2_W, fc2_b, fc3_W, fc3_b, fc4_W, fc4_b):
    src = edge_index[0]
    dst = edge_index[1]
    src3 = src.reshape(N_TILES, NCH, EC)
    dst3 = dst.reshape(N_TILES, NCH, EC)

    zeros16 = jnp.zeros((N_NODES, DEGW), jnp.float32)
    deg_parts = _sc_deg(src3, zeros16)
    deg = deg_parts[0, :, 0] + deg_parts[1, :, 0]
    dinv = jnp.where(deg > 0, 1.0 / jnp.sqrt(jnp.maximum(deg, 1e-12)), 0.0)
    s = dinv[:, None]

    h = _layer(x, W1, b1, g1, be1, s, src3, dst3)
    h = _layer(h, W2, b2, g2, be2, s, src3, dst3)
    h = _layer(h, W3, b3, g3, be3, s, src3, dst3)

    h3 = h.reshape(-1, NUM_NODES_G, 256)
    return _edge_mlp(h3, pair_src, pair_dst, fc1_W, fc1_b, fc2_W, fc2_b,
                     fc3_W, fc3_b, fc4_W, fc4_b)


# speedup vs baseline: 7.6179x; 1.0192x over previous
import functools

import jax
import jax.numpy as jnp
from jax import lax
from jax.experimental import pallas as pl
from jax.experimental.pallas import tpu as pltpu
from jax.experimental.pallas import tpu_sc as plsc

N_NODES = 30720
E_EDGES = 491520
N_TILES = 16
N_CORES = 2
EC = 128
EPT = E_EDGES // N_TILES
NCH = EPT // EC
IBLK = 48
NBLK = NCH // IBLK
NSLOT = 12
LOOKAHEAD = 6
RPT = N_NODES // N_TILES
FC = 32


def _sc_prop_body(C, u_hbm, src_hbm, dst_hbm, r_hbm,
                  acc, src_idx, dst_idx, rows, zbuf, gsems, ssems):
    cid = lax.axis_index("c")
    sid = lax.axis_index("s")

    def _zb(i, _):
        for j in range(FC // 16):
            zbuf[i, pl.ds(j * 16, 16)] = jnp.zeros((16,), jnp.float32)
        return 0
    lax.fori_loop(0, 128, _zb, 0)

    rowbase = sid * RPT
    for rr in range(C // N_CORES):
        chunk = rr * N_CORES + cid
        uv = u_hbm.at[chunk]
        rv = r_hbm.at[chunk]

        def _zero(m, _):
            pltpu.sync_copy(zbuf, acc.at[pl.ds(rowbase + m * 128, 128)])
            return 0
        lax.fori_loop(0, RPT // 128, _zero, 0)
        plsc.subcore_barrier()

        for blk in range(NBLK):
            pltpu.sync_copy(src_hbm.at[sid, pl.ds(blk * IBLK, IBLK)], src_idx)
            pltpu.sync_copy(dst_hbm.at[sid, pl.ds(blk * IBLK, IBLK)], dst_idx)
            for b in range(LOOKAHEAD):
                pltpu.async_copy(uv.at[src_idx.at[b]], rows.at[b], gsems.at[b])

            def _body(g, _):
                for u in range(NSLOT):
                    j = g * NSLOT + u
                    b = u
                    pltpu.make_async_copy(
                        uv.at[src_idx.at[j]], rows.at[b], gsems.at[b]).wait()
                    pltpu.async_copy(rows.at[b], acc.at[dst_idx.at[j]],
                                     ssems.at[b], add=True)
                    nxt = j + LOOKAHEAD
                    nb = (u + LOOKAHEAD) % NSLOT

                    @pl.when(nxt < IBLK)
                    def _():
                        @pl.when(j >= LOOKAHEAD)
                        def _():
                            pltpu.make_async_copy(
                                rows.at[nb],
                                acc.at[dst_idx.at[j - LOOKAHEAD]],
                                ssems.at[nb]).wait()
                        pltpu.async_copy(uv.at[src_idx.at[nxt]], rows.at[nb],
                                         gsems.at[nb])
                return 0
            lax.fori_loop(0, IBLK // NSLOT, _body, 0)
            for k in range(NSLOT):
                j = IBLK - NSLOT + k
                b = j % NSLOT
                pltpu.make_async_copy(
                    rows.at[b], acc.at[dst_idx.at[j]], ssems.at[b]).wait()
        plsc.subcore_barrier()

        def _out(m, _):
            sl = pl.ds(rowbase + m * 128, 128)
            pltpu.sync_copy(acc.at[sl], rows.at[0])
            pltpu.sync_copy(rows.at[0], rv.at[sl])
            return 0
        lax.fori_loop(0, RPT // 128, _out, 0)
        plsc.subcore_barrier()


@functools.partial(jax.jit, static_argnums=(3,))
def _sc_prop(u3, src3, dst3, C):
    mesh = plsc.VectorSubcoreMesh(core_axis_name="c", subcore_axis_name="s")
    return pl.kernel(
        functools.partial(_sc_prop_body, C),
        out_type=jax.ShapeDtypeStruct((C, N_NODES, FC), jnp.float32),
        mesh=mesh,
        compiler_params=pltpu.CompilerParams(use_tc_tiling_on_sc=False),
        scratch_types=[
            pltpu.VMEM_SHARED((N_NODES, FC), jnp.float32),
            pltpu.VMEM((IBLK, EC), jnp.int32),
            pltpu.VMEM((IBLK, EC), jnp.int32),
            pltpu.VMEM((NSLOT, EC, FC), jnp.float32),
            pltpu.VMEM((128, FC), jnp.float32),
            pltpu.SemaphoreType.DMA((NSLOT,)),
            pltpu.SemaphoreType.DMA((NSLOT,)),
        ],
    )(u3, src3, dst3)


def _R(u, src3, dst3):
    C = u.shape[1] // FC
    u3 = u.reshape(N_NODES, C, FC).transpose(1, 0, 2)
    r3 = _sc_prop(u3, src3, dst3, C)
    return r3.transpose(1, 0, 2).reshape(N_NODES, C * FC)


DEGW = 16
NCH_CORE = NCH // N_CORES
DEG_IBLK = 40
DEG_NSLOT = 8


def _sc_deg_body(src_hbm, deg_hbm, acc, idx, ones, ssems):
    cid = lax.axis_index("c")
    sid = lax.axis_index("s")

    def _ob(i, _):
        ones[i, pl.ds(0, 16)] = jnp.full((16,), 1.0, jnp.float32)
        return 0
    lax.fori_loop(0, EC, _ob, 0)

    rowbase = sid * RPT
    plsc.subcore_barrier()

    for blk in range(NCH_CORE // DEG_IBLK):
        base = cid * NCH_CORE + blk * DEG_IBLK
        pltpu.sync_copy(src_hbm.at[sid, pl.ds(base, DEG_IBLK)], idx)

        def _body(g, _):
            for u in range(DEG_NSLOT):
                j = g * DEG_NSLOT + u
                pltpu.async_copy(ones, acc.at[idx.at[j]], ssems.at[u],
                                 add=True)

                @pl.when(j >= DEG_NSLOT)
                def _():
                    pltpu.make_async_copy(
                        ones, acc.at[idx.at[j - DEG_NSLOT]], ssems.at[u]).wait()
            return 0
        lax.fori_loop(0, DEG_IBLK // DEG_NSLOT, _body, 0)
        for k in range(DEG_NSLOT):
            j = DEG_IBLK - DEG_NSLOT + k
            pltpu.make_async_copy(
                ones, acc.at[idx.at[j]], ssems.at[j % DEG_NSLOT]).wait()
    plsc.subcore_barrier()

    def _out(m, _):
        sl = pl.ds(rowbase + m * 128, 128)
        pltpu.sync_copy(acc.at[sl], ones)
        pltpu.sync_copy(ones, deg_hbm.at[cid].at[sl])
        return 0
    lax.fori_loop(0, RPT // 128, _out, 0)


@jax.jit
def _sc_deg(src3, zeros16):
    mesh = plsc.VectorSubcoreMesh(core_axis_name="c", subcore_axis_name="s")

    def body(src_hbm, z_hbm, deg_hbm, acc, idx, ones, ssems):
        sid = lax.axis_index("s")
        rowbase = sid * RPT
        pltpu.sync_copy(z_hbm.at[pl.ds(rowbase, RPT)],
                        acc.at[pl.ds(rowbase, RPT)])
        _sc_deg_body(src_hbm, deg_hbm, acc, idx, ones, ssems)

    return pl.kernel(
        body,
        out_type=jax.ShapeDtypeStruct((N_CORES, N_NODES, DEGW), jnp.float32),
        mesh=mesh,
        compiler_params=pltpu.CompilerParams(use_tc_tiling_on_sc=False),
        scratch_types=[
            pltpu.VMEM_SHARED((N_NODES, DEGW), jnp.float32),
            pltpu.VMEM((DEG_IBLK, EC), jnp.int32),
            pltpu.VMEM((EC, DEGW), jnp.float32),
            pltpu.SemaphoreType.DMA((DEG_NSLOT,)),
        ],
    )(src3, zeros16)


BN = 512
NBLK_TC = N_NODES // BN


def _row_specs(Fin):
    return [pl.BlockSpec((BN, Fin), lambda i: (i, 0))]


def _start_body(h_ref, w_ref, s_ref, acc_ref, u_ref):
    h = h_ref[...]
    acc_ref[...] = jnp.dot(h, w_ref[...], preferred_element_type=jnp.float32)
    u_ref[...] = h * s_ref[...]


def _tc_start(h, W0, s):
    Fin = h.shape[1]
    return pl.pallas_call(
        _start_body,
        grid=(NBLK_TC,),
        in_specs=[
            pl.BlockSpec((BN, Fin), lambda i: (i, 0)),
            pl.BlockSpec((Fin, 256), lambda i: (0, 0)),
            pl.BlockSpec((BN, 1), lambda i: (i, 0)),
        ],
        out_specs=[
            pl.BlockSpec((BN, 256), lambda i: (i, 0)),
            pl.BlockSpec((BN, Fin), lambda i: (i, 0)),
        ],
        out_shape=[
            jax.ShapeDtypeStruct((N_NODES, 256), jnp.float32),
            jax.ShapeDtypeStruct((N_NODES, Fin), jnp.float32),
        ],
    )(h, W0, s)


def _step1_body(r_ref, s_ref, w_ref, acc_ref, acc_out, t_out, u_out):
    t = -(s_ref[...] * r_ref[...])
    acc_out[...] = acc_ref[...] + jnp.dot(
        t, w_ref[...], preferred_element_type=jnp.float32)
    t_out[...] = t
    u_out[...] = t * s_ref[...]


def _stepk_body(r_ref, tp_ref, s_ref, w_ref, acc_ref, acc_out, t_out, u_out):
    t = -2.0 * (s_ref[...] * r_ref[...]) - tp_ref[...]
    acc_out[...] = acc_ref[...] + jnp.dot(
        t, w_ref[...], preferred_element_type=jnp.float32)
    t_out[...] = t
    u_out[...] = t * s_ref[...]


def _tc_step(r, tprev, s, Wk, acc):
    Fin = r.shape[1]
    row = lambda F: pl.BlockSpec((BN, F), lambda i: (i, 0))
    ins = [r] if tprev is None else [r, tprev]
    in_specs = [row(Fin)] * len(ins) + [
        pl.BlockSpec((BN, 1), lambda i: (i, 0)),
        pl.BlockSpec((Fin, 256), lambda i: (0, 0)),
        row(256),
    ]
    body = _step1_body if tprev is None else _stepk_body
    return pl.pallas_call(
        body,
        grid=(NBLK_TC,),
        in_specs=in_specs,
        out_specs=[row(256), row(Fin), row(Fin)],
        out_shape=[
            jax.ShapeDtypeStruct((N_NODES, 256), jnp.float32),
            jax.ShapeDtypeStruct((N_NODES, Fin), jnp.float32),
            jax.ShapeDtypeStruct((N_NODES, Fin), jnp.float32),
        ],
    )(*ins, s, Wk, acc)


def _final_body(r_ref, tp_ref, s_ref, w_ref, acc_ref, b_ref, g_ref, be_ref,
                h_out):
    t = -2.0 * (s_ref[...] * r_ref[...]) - tp_ref[...]
    o = acc_ref[...] + jnp.dot(
        t, w_ref[...], preferred_element_type=jnp.float32) + b_ref[...]
    o = jnp.where(o >= 0, o, 0.01 * o)
    mu = jnp.mean(o, axis=-1, keepdims=True)
    var = jnp.mean((o - mu) ** 2, axis=-1, keepdims=True)
    h_out[...] = (o - mu) / jnp.sqrt(var + 1e-5) * g_ref[...] + be_ref[...]


def _tc_final(r, tprev, s, Wk, acc, b, g, be):
    Fin = r.shape[1]
    row = lambda F: pl.BlockSpec((BN, F), lambda i: (i, 0))
    vec = pl.BlockSpec((1, 256), lambda i: (0, 0))
    return pl.pallas_call(
        _final_body,
        grid=(NBLK_TC,),
        in_specs=[row(Fin), row(Fin),
                  pl.BlockSpec((BN, 1), lambda i: (i, 0)),
                  pl.BlockSpec((Fin, 256), lambda i: (0, 0)),
                  row(256), vec, vec, vec],
        out_specs=row(256),
        out_shape=jax.ShapeDtypeStruct((N_NODES, 256), jnp.float32),
    )(r, tprev, s, Wk, acc, b.reshape(1, -1), g.reshape(1, -1),
      be.reshape(1, -1))


def _layer(h, W, b, g, be, s, src3, dst3):
    K = W.shape[0]
    acc, u = _tc_start(h, W[0], s)
    r = _R(u, src3, dst3)
    acc, T1, u = _tc_step(r, None, s, W[1], acc)
    tm2, tm1 = h, T1
    for k in range(2, K):
        r = _R(u, src3, dst3)
        if k < K - 1:
            acc, tk, u = _tc_step(r, tm2, s, W[k], acc)
            tm2, tm1 = tm1, tk
        else:
            return _tc_final(r, tm2, s, W[k], acc, b, g, be)


GB = 32
NUM_PAIRS = 90
NUM_NODES_G = 30


def _mlp_body(h_ref, ps_ref, pd_ref, w1a_ref, w1b_ref, b1_ref, w2_ref,
              b2_ref, w3_ref, b3_ref, w4_ref, b4_ref, out_ref,
              p1_s, p2_s, o1_s):
    hb = h_ref[...].reshape(GB * NUM_NODES_G, 256)
    p1_s[...] = jnp.dot(hb, w1a_ref[...],
                        preferred_element_type=jnp.float32).reshape(
        GB, NUM_NODES_G, 256)
    p2_s[...] = jnp.dot(hb, w1b_ref[...],
                        preferred_element_type=jnp.float32).reshape(
        GB, NUM_NODES_G, 256)
    for j in range(NUM_PAIRS):
        rs = ps_ref[j]
        rd = pd_ref[j]
        e1 = p1_s[:, pl.ds(rs, 1), :].reshape(GB, 256)
        e2 = p2_s[:, pl.ds(rd, 1), :].reshape(GB, 256)
        o1_s[j] = e1 + e2
    o = o1_s[...] + b1_ref[...]
    o = jnp.where(o >= 0, o, 0.01 * o)
    o = o.reshape(NUM_PAIRS * GB, 256)
    o = jnp.dot(o, w2_ref[...], preferred_element_type=jnp.float32) + b2_ref[...]
    o = jnp.where(o >= 0, o, 0.01 * o)
    o = jnp.dot(o, w3_ref[...], preferred_element_type=jnp.float32) + b3_ref[...]
    o = jnp.where(o >= 0, o, 0.01 * o)
    o = jnp.dot(o, w4_ref[...], preferred_element_type=jnp.float32) + b4_ref[...]
    out_ref[...] = o.reshape(NUM_PAIRS, GB).T


def _edge_mlp(h3, pair_src, pair_dst, fc1_W, fc1_b, fc2_W, fc2_b,
              fc3_W, fc3_b, fc4_W, fc4_b):
    B = h3.shape[0]
    grid = (B // GB,)
    full = lambda *sh: pl.BlockSpec(sh, lambda i: (0,) * len(sh))
    out2d = pl.pallas_call(
        _mlp_body,
        grid=grid,
        in_specs=[
            pl.BlockSpec((GB, NUM_NODES_G, 256), lambda i: (i, 0, 0)),
            pl.BlockSpec(memory_space=pltpu.SMEM),
            pl.BlockSpec(memory_space=pltpu.SMEM),
            full(256, 256), full(256, 256), full(1, 256),
            full(256, 128), full(1, 128),
            full(128, 128), full(1, 128),
            full(128, 1), full(1, 1),
        ],
        out_specs=pl.BlockSpec((GB, NUM_PAIRS), lambda i: (i, 0)),
        out_shape=jax.ShapeDtypeStruct((B, NUM_PAIRS), jnp.float32),
        scratch_shapes=[
            pltpu.VMEM((GB, NUM_NODES_G, 256), jnp.float32),
            pltpu.VMEM((GB, NUM_NODES_G, 256), jnp.float32),
            pltpu.VMEM((NUM_PAIRS, GB, 256), jnp.float32),
        ],
    )(h3, pair_src, pair_dst, fc1_W[:256], fc1_W[256:],
      fc1_b.reshape(1, -1), fc2_W, fc2_b.reshape(1, -1),
      fc3_W, fc3_b.reshape(1, -1), fc4_W, fc4_b.reshape(1, -1))
    return out2d.reshape(-1)


def kernel(x, edge_index, pair_src, pair_dst, W1, b1, g1, be1, W2, b2, g2, be2,
           W3, b3, g3, be3, fc1_W, fc1_b, fc2_W, fc2_b, fc3_W, fc3_b, fc4_W, fc4_b):
    src = edge_index[0]
    dst = edge_index[1]
    src3 = src.reshape(N_TILES, NCH, EC)
    dst3 = dst.reshape(N_TILES, NCH, EC)

    zeros16 = jnp.zeros((N_NODES, DEGW), jnp.float32)
    deg_parts = _sc_deg(src3, zeros16)
    deg = deg_parts[0, :, 0] + deg_parts[1, :, 0]
    dinv = jnp.where(deg > 0, 1.0 / jnp.sqrt(jnp.maximum(deg, 1e-12)), 0.0)
    s = dinv[:, None]

    h = _layer(x, W1, b1, g1, be1, s, src3, dst3)
    h = _layer(h, W2, b2, g2, be2, s, src3, dst3)
    h = _layer(h, W3, b3, g3, be3, s, src3, dst3)

    h3 = h.reshape(-1, NUM_NODES_G, 256)
    return _edge_mlp(h3, pair_src, pair_dst, fc1_W, fc1_b, fc2_W, fc2_b,
                     fc3_W, fc3_b, fc4_W, fc4_b)
